# baseline (speedup 1.0000x reference)
import jax
import jax.numpy as jnp
from jax.experimental import pallas as pl
from jax.experimental.pallas import tpu as pltpu

_T = 512
_H = 768
_DFF = 768
_E = 64
_EPS = 1e-6

_B = 32
_G = 96


def _router_sort_kernel(x_ref, rw_ref, gw_ref, gb_ref,
                        tf_ref, stok_ref, scoef_ref, bsn_ref):
    x = x_ref[...]
    var = jnp.mean(x * x, axis=1, keepdims=True)
    t = x * jax.lax.rsqrt(var + _EPS) * rw_ref[...]
    tf_ref[...] = t

    logits = jax.lax.dot_general(
        t, gw_ref[...], (((1,), (1,)), ((), ())),
        preferred_element_type=jnp.float32) + gb_ref[...]
    iota_e = jax.lax.broadcasted_iota(jnp.int32, (_T, _E), 1)
    m1 = jnp.max(logits, axis=1, keepdims=True)
    i1 = jnp.min(jnp.where(logits == m1, iota_e, _E), axis=1, keepdims=True)
    l2 = jnp.where(iota_e == i1, -jnp.inf, logits)
    m2 = jnp.max(l2, axis=1, keepdims=True)
    i2 = jnp.min(jnp.where(l2 == m2, iota_e, _E), axis=1, keepdims=True)
    a = jnp.exp(m2 - m1)
    w1c = 1.0 / (1.0 + a)
    w2c = a / (1.0 + a)

    oh0 = (iota_e == i1).astype(jnp.float32)
    oh1 = (iota_e == i2).astype(jnp.float32)
    cnt = oh0 + oh1

    tri = (jax.lax.broadcasted_iota(jnp.int32, (_T, _T), 1) <
           jax.lax.broadcasted_iota(jnp.int32, (_T, _T), 0)).astype(jnp.float32)
    csum = jax.lax.dot_general(tri, cnt, (((1,), (0,)), ((), ())),
                               preferred_element_type=jnp.float32)

    ones_t = jnp.ones((_T, 1), jnp.float32)
    tot = jax.lax.dot_general(cnt, ones_t, (((0,), (0,)), ((), ())),
                              preferred_element_type=jnp.float32)
    nb = jnp.floor((tot + (_B - 1)) / _B)
    lower = (jax.lax.broadcasted_iota(jnp.int32, (_E, _E), 1) <=
             jax.lax.broadcasted_iota(jnp.int32, (_E, _E), 0)).astype(jnp.float32)
    bend = jax.lax.dot_general(lower, nb, (((1,), (0,)), ((), ())),
                               preferred_element_type=jnp.float32)
    bstart = bend - nb

    poff0 = _B * jax.lax.dot_general(oh0, bstart, (((1,), (0,)), ((), ())),
                                     preferred_element_type=jnp.float32)
    poff1 = _B * jax.lax.dot_general(oh1, bstart, (((1,), (0,)), ((), ())),
                                     preferred_element_type=jnp.float32)
    pos0 = poff0 + jnp.sum(oh0 * csum, axis=1, keepdims=True)
    pos1 = poff1 + jnp.sum(oh1 * csum, axis=1, keepdims=True)
    q0 = jnp.floor(pos0 * (1.0 / _B))
    r0 = pos0 - _B * q0
    q1 = jnp.floor(pos1 * (1.0 / _B))
    r1 = pos1 - _B * q1

    iota_q = jax.lax.broadcasted_iota(jnp.int32, (_T, _G), 1)
    iota_r = jax.lax.broadcasted_iota(jnp.int32, (_T, _B), 1)
    mq0 = (iota_q == q0.astype(jnp.int32)).astype(jnp.float32)
    mq1 = (iota_q == q1.astype(jnp.int32)).astype(jnp.float32)
    mr0 = (iota_r == r0.astype(jnp.int32)).astype(jnp.float32)
    mr1 = (iota_r == r1.astype(jnp.int32)).astype(jnp.float32)
    tok = jax.lax.broadcasted_iota(jnp.int32, (_T, 1), 0)
    tok_lo = (tok % 256).astype(jnp.float32)
    tok_hi = (tok // 256).astype(jnp.float32)

    def sc2(lhs, rhs):
        return jax.lax.dot_general(lhs, rhs, (((0,), (0,)), ((), ())),
                                   preferred_element_type=jnp.float32)

    stok_ref[...] = (sc2(mq0, mr0 * tok_lo) + 256.0 * sc2(mq0, mr0 * tok_hi) +
                     sc2(mq1, mr1 * tok_lo) + 256.0 * sc2(mq1, mr1 * tok_hi))
    scoef_ref[...] = sc2(mq0, mr0 * w1c) + sc2(mq1, mr1 * w2c)

    ones_e = jnp.ones((_E, 1), jnp.float32)
    bs_row = jax.lax.dot_general(ones_e, bstart, (((1,), (1,)), ((), ())),
                                 preferred_element_type=jnp.float32)[0:1]
    nb_row = jax.lax.dot_general(ones_e, nb, (((1,), (1,)), ((), ())),
                                 preferred_element_type=jnp.float32)[0:1]
    bsn_ref[:, :_E] = bs_row.astype(jnp.int32)
    bsn_ref[:, _E:] = nb_row.astype(jnp.int32)


_EPG = 4


def _moe_kernel(bsn_sref, stok_ref, scoef_ref, tf_ref, x_ref,
                w1_ref, w3_ref, w2_ref, o_ref):
    g = pl.program_id(0)

    @pl.when(g == 0)
    def _():
        o_ref[...] = x_ref[...]

    iota_tb = jax.lax.broadcasted_iota(jnp.int32, (_T, _B), 0)

    for j in range(_EPG):
        e = g * _EPG + j
        bs = bsn_sref[0, e]
        nbe = bsn_sref[0, _E + e]

        def body(i, carry, _j=j, _bs=bs):
            row = pl.ds(_bs + i, 1)
            st = stok_ref[row, :]
            sc = scoef_ref[row, :]
            pt = (iota_tb == st.astype(jnp.int32)).astype(jnp.float32)
            tb = jax.lax.dot_general(pt, tf_ref[...],
                                     (((0,), (0,)), ((), ())),
                                     preferred_element_type=jnp.float32)
            gg = jax.lax.dot_general(tb, w1_ref[_j],
                                     (((1,), (1,)), ((), ())),
                                     preferred_element_type=jnp.float32)
            uu = jax.lax.dot_general(tb, w3_ref[_j],
                                     (((1,), (1,)), ((), ())),
                                     preferred_element_type=jnp.float32)
            h = (gg * jax.lax.logistic(gg)) * uu
            y = jax.lax.dot_general(h, w2_ref[_j],
                                    (((1,), (1,)), ((), ())),
                                    preferred_element_type=jnp.float32)
            contrib = jax.lax.dot_general(pt * sc, y,
                                          (((1,), (0,)), ((), ())),
                                          preferred_element_type=jnp.float32)
            o_ref[...] += contrib
            return carry

        jax.lax.fori_loop(0, nbe, body, 0)


def kernel(x, rms_weight, gate_w, gate_b, w1, w3, w2):
    tf, stok, scoef, bsn = pl.pallas_call(
        _router_sort_kernel,
        out_shape=(
            jax.ShapeDtypeStruct((_T, _H), jnp.float32),
            jax.ShapeDtypeStruct((_G, _B), jnp.float32),
            jax.ShapeDtypeStruct((_G, _B), jnp.float32),
            jax.ShapeDtypeStruct((1, 2 * _E), jnp.int32),
        ),
    )(x, rms_weight.reshape(1, _H), gate_w, gate_b.reshape(1, _E))

    grid_spec = pltpu.PrefetchScalarGridSpec(
        num_scalar_prefetch=1,
        grid=(_E // _EPG,),
        in_specs=[
            pl.BlockSpec((_G, _B), lambda g, s: (0, 0)),
            pl.BlockSpec((_G, _B), lambda g, s: (0, 0)),
            pl.BlockSpec((_T, _H), lambda g, s: (0, 0)),
            pl.BlockSpec((_T, _H), lambda g, s: (0, 0)),
            pl.BlockSpec((_EPG, _DFF, _H), lambda g, s: (g, 0, 0)),
            pl.BlockSpec((_EPG, _DFF, _H), lambda g, s: (g, 0, 0)),
            pl.BlockSpec((_EPG, _H, _DFF), lambda g, s: (g, 0, 0)),
        ],
        out_specs=pl.BlockSpec((_T, _H), lambda g, s: (0, 0)),
    )
    out = pl.pallas_call(
        _moe_kernel,
        grid_spec=grid_spec,
        out_shape=jax.ShapeDtypeStruct((_T, _H), jnp.float32),
        compiler_params=pltpu.CompilerParams(
            dimension_semantics=("arbitrary",),
            vmem_limit_bytes=120 * 1024 * 1024),
    )(bsn, stok, scoef, tf, x, w1, w3, w2)
    return out
